# baseline (speedup 1.0000x reference)
import jax
import jax.numpy as jnp
from jax import lax
from jax.experimental import pallas as pl


def _gate_kernel(x_ref, wg_ref, w_ref):
    x = x_ref[...]
    wg = wg_ref[...]
    logits = lax.dot_general(x, wg, (((1,), (1,)), ((), ())),
                             preferred_element_type=jnp.float32)
    m = jnp.max(logits, axis=1, keepdims=True)
    p = jnp.exp(logits - m)
    g = p / jnp.sum(p, axis=1, keepdims=True)
    num_e = g.shape[1]
    iota = lax.broadcasted_iota(jnp.int32, g.shape, 1)
    v1 = jnp.max(g, axis=1, keepdims=True)
    i1 = jnp.min(jnp.where(g >= v1, iota, num_e), axis=1, keepdims=True)
    g2 = jnp.where(iota == i1, -1.0, g)
    v2 = jnp.max(g2, axis=1, keepdims=True)
    i2 = jnp.min(jnp.where(g2 >= v2, iota, num_e), axis=1, keepdims=True)
    keep = (iota == i1) | (iota == i2)
    w_ref[...] = jnp.where(keep, g, 0.0) / (v1 + v2 + 1e-9)


def _ffn_kernel(w8_ref, x_ref, w1_ref, w2_ref, w3_ref, out_ref):
    e = pl.program_id(0)
    hb = pl.program_id(1)

    @pl.when((e == 0) & (hb == 0))
    def _():
        out_ref[...] = jnp.zeros_like(out_ref)

    x = x_ref[...]
    w1 = w1_ref[0]
    w2 = w2_ref[0]
    w3 = w3_ref[0]
    h1 = lax.dot_general(x, w1, (((1,), (1,)), ((), ())),
                         preferred_element_type=jnp.float32)
    h2 = lax.dot_general(x, w2, (((1,), (1,)), ((), ())),
                         preferred_element_type=jnp.float32)
    s = 1.0 / (1.0 + jnp.exp(-h1))
    gmid = h1 * s * h2
    part = lax.dot_general(gmid, w3, (((1,), (1,)), ((), ())),
                           preferred_element_type=jnp.float32)
    w8 = w8_ref[...]
    onehot = (lax.broadcasted_iota(jnp.int32, w8.shape, 1) == e)
    wcol = jnp.sum(jnp.where(onehot, w8, 0.0), axis=1, keepdims=True)
    out_ref[...] += wcol * part


def kernel(x, Wg, W1, W2, W3):
    n_tok, d_model = x.shape
    num_e, hidden, _ = W1.shape
    hb_sz = 384
    nhb = hidden // hb_sz

    w8 = pl.pallas_call(
        _gate_kernel,
        out_shape=jax.ShapeDtypeStruct((n_tok, num_e), jnp.float32),
    )(x, Wg)

    out = pl.pallas_call(
        _ffn_kernel,
        grid=(num_e, nhb),
        in_specs=[
            pl.BlockSpec((n_tok, num_e), lambda e, hb: (0, 0)),
            pl.BlockSpec((n_tok, d_model), lambda e, hb: (0, 0)),
            pl.BlockSpec((1, hb_sz, d_model), lambda e, hb: (e, hb, 0)),
            pl.BlockSpec((1, hb_sz, d_model), lambda e, hb: (e, hb, 0)),
            pl.BlockSpec((1, d_model, hb_sz), lambda e, hb: (e, 0, hb)),
        ],
        out_specs=pl.BlockSpec((n_tok, d_model), lambda e, hb: (0, 0)),
        out_shape=jax.ShapeDtypeStruct((n_tok, d_model), jnp.float32),
    )(w8, x, W1, W2, W3)
    return out


# speedup vs baseline: 1.2057x; 1.2057x over previous
import jax
import jax.numpy as jnp
from jax import lax
from jax.experimental import pallas as pl
from jax.experimental.pallas import tpu as pltpu
from jax.experimental.pallas import tpu_sc as plsc

_NC = 2
_NS = 16
_BLK = 512
_MAXB = 16
_HB = 512


def _gate_kernel(x_ref, wg_ref, i1_ref, i2_ref, w1_ref, w2_ref):
    x = x_ref[...]
    wg = wg_ref[...]
    logits = lax.dot_general(x, wg, (((1,), (1,)), ((), ())),
                             preferred_element_type=jnp.float32)
    m = jnp.max(logits, axis=1, keepdims=True)
    p = jnp.exp(logits - m)
    g = p / jnp.sum(p, axis=1, keepdims=True)
    num_e = g.shape[1]
    iota = lax.broadcasted_iota(jnp.int32, g.shape, 1)
    v1 = jnp.max(g, axis=1, keepdims=True)
    i1 = jnp.min(jnp.where(g >= v1, iota, num_e), axis=1, keepdims=True)
    g2 = jnp.where(iota == i1, -1.0, g)
    v2 = jnp.max(g2, axis=1, keepdims=True)
    i2 = jnp.min(jnp.where(g2 >= v2, iota, num_e), axis=1, keepdims=True)
    denom = v1 + v2 + 1e-9
    i1_ref[...] = i1
    i2_ref[...] = i2
    w1_ref[...] = v1 / denom
    w2_ref[...] = v2 / denom


def _vgather(src, idx):
    return src.at[idx].get(mode="promise_in_bounds")


_L15 = None


def _splat_last(cs):
    return _vgather(cs, jnp.full((16,), 15, jnp.int32))


def _lane_hist(tv, lanes):
    cnt = jnp.zeros((16,), jnp.int32)
    for e in range(8):
        cs = plsc.cumsum(jnp.where(tv == e, 1, 0))
        cnt = cnt + jnp.where(lanes == e, _splat_last(cs), 0)
    return cnt


def _route_body(eflat, wkflat, x, xs, wslot, meta, slot_e, slot_o,
                tidx2_v, countall_v, shared_counts, cnta_v, cntb_v,
                xv, slots_v, wkv, bev, semx, sem1, sem2):
    c = lax.axis_index("c")
    s = lax.axis_index("s")
    w_rank = 2 * s + c
    ca = 2 * s
    lanes = lax.broadcasted_iota(jnp.int32, (16,), 0)

    tok_base = (w_rank % 16) * 128
    dx = pltpu.async_copy(x.at[pl.ds(tok_base, 128)], xv, semx)

    pltpu.sync_copy(eflat.at[pl.ds(ca * 128, 256)], tidx2_v)

    cnt_a = jnp.zeros((16,), jnp.int32)
    cnt_b = jnp.zeros((16,), jnp.int32)
    for v in range(8):
        cnt_a = cnt_a + _lane_hist(tidx2_v[pl.ds(v * 16, 16)], lanes)
        cnt_b = cnt_b + _lane_hist(tidx2_v[pl.ds(128 + v * 16, 16)], lanes)
    cnta_v[...] = cnt_a
    cntb_v[...] = cnt_b

    pltpu.sync_copy(cnta_v, shared_counts.at[pl.ds(ca * 16, 16)])
    pltpu.sync_copy(cntb_v, shared_counts.at[pl.ds(ca * 16 + 16, 16)])
    plsc.subcore_barrier()
    pltpu.sync_copy(shared_counts, countall_v)

    def accrow(w, carry):
        tot, pre = carry
        row = countall_v[pl.ds(w * 16, 16)]
        return tot + row, pre + jnp.where(w < w_rank, row, 0)

    zero = jnp.zeros((16,), jnp.int32)
    tot, pre = lax.fori_loop(0, 32, accrow, (zero, zero))

    nb_vec = (tot + (_BLK - 1)) // _BLK
    inc = plsc.cumsum(nb_vec)
    excl = inc - nb_vec
    base_vec = excl * _BLK + pre

    bev_vec = jnp.zeros((16,), jnp.int32)
    for e in range(8):
        ince = _vgather(inc, jnp.full((16,), e, jnp.int32))
        bev_vec = bev_vec + jnp.where(ince <= lanes, 1, 0)
    nblocks = _vgather(inc, jnp.full((16,), 7, jnp.int32))
    tail_e = _vgather(bev_vec, jnp.maximum(nblocks - 1, 0))
    bev[pl.ds(0, 16)] = jnp.minimum(bev_vec, tail_e)
    bev[pl.ds(16, 16)] = nblocks

    off = c * 128
    for v in range(8):
        tv = tidx2_v[pl.ds(off + v * 16, 16)]
        rank = jnp.zeros((16,), jnp.int32)
        cnt = jnp.zeros((16,), jnp.int32)
        for e in range(8):
            m = tv == e
            cs = plsc.cumsum(jnp.where(m, 1, 0))
            rank = rank + jnp.where(m, cs - 1, 0)
            cnt = cnt + jnp.where(lanes == e, _splat_last(cs), 0)
        sel = _vgather(base_vec, tv)
        slots_v[pl.ds(v * 16, 16)] = sel + rank
        base_vec = base_vec + cnt

    pltpu.sync_copy(wkflat.at[pl.ds(w_rank * 128, 128)], wkv)
    dx.wait()
    d1 = pltpu.async_copy(xv, xs.at[slots_v], sem1)
    d2 = pltpu.async_copy(wkv, wslot.at[slots_v], sem2)

    @pl.when(w_rank < 16)
    def _():
        pltpu.sync_copy(slots_v, slot_e.at[pl.ds(tok_base, 128)])

    @pl.when(w_rank >= 16)
    def _():
        pltpu.sync_copy(slots_v, slot_o.at[pl.ds(tok_base, 128)])

    @pl.when((c == 0) & (s == 0))
    def _():
        pltpu.sync_copy(bev, meta)

    d1.wait()
    d2.wait()


def _ffn_kernel(be_ref, nb_ref, xs_ref, ws_ref, w1_ref, w2_ref, w3_ref,
                ys_ref):
    b = pl.program_id(0)
    hb = pl.program_id(1)
    nhb = pl.num_programs(1)

    @pl.when(b < nb_ref[0])
    def _():
        xs = xs_ref[...]
        w1 = w1_ref[0]
        w2 = w2_ref[0]
        w3 = w3_ref[0]
        h1 = lax.dot_general(xs, w1, (((1,), (1,)), ((), ())),
                             preferred_element_type=jnp.float32)
        h2 = lax.dot_general(xs, w2, (((1,), (1,)), ((), ())),
                             preferred_element_type=jnp.float32)
        sg = 1.0 / (1.0 + jnp.exp(-h1))
        gmid = h1 * sg * h2
        part = lax.dot_general(gmid, w3, (((1,), (1,)), ((), ())),
                               preferred_element_type=jnp.float32)

        @pl.when(hb == 0)
        def _():
            ys_ref[...] = part

        @pl.when(hb != 0)
        def _():
            ys_ref[...] += part

        @pl.when(hb == nhb - 1)
        def _():
            ys_ref[...] *= ws_ref[...]


def _combine_body(ys, slot_e, slot_o, out, se_v, so_v, ya, yb, sem1, sem2):
    c = lax.axis_index("c")
    s = lax.axis_index("s")
    w = 2 * s + c
    base = w * 64
    pltpu.sync_copy(slot_e.at[pl.ds(base, 64)], se_v)
    pltpu.sync_copy(slot_o.at[pl.ds(base, 64)], so_v)
    g1 = pltpu.async_copy(ys.at[se_v], ya, sem1)
    g2 = pltpu.async_copy(ys.at[so_v], yb, sem2)
    g1.wait()
    g2.wait()

    def add_row(j, _):
        def add_vec(v, _):
            sl = pl.ds(v * 16, 16)
            ya[j, sl] = ya[j, sl] + yb[j, sl]
            return 0
        lax.fori_loop(0, 48, add_vec, 0)
        return 0
    lax.fori_loop(0, 64, add_row, 0)
    pltpu.sync_copy(ya, out.at[pl.ds(base, 64)])


def kernel(x, Wg, W1, W2, W3):
    n_tok, d_model = x.shape
    num_e, hidden, _ = W1.shape
    nhb = hidden // _HB
    n_slot = _MAXB * _BLK

    i1, i2, w1n, w2n = pl.pallas_call(
        _gate_kernel,
        out_shape=[
            jax.ShapeDtypeStruct((n_tok, 1), jnp.int32),
            jax.ShapeDtypeStruct((n_tok, 1), jnp.int32),
            jax.ShapeDtypeStruct((n_tok, 1), jnp.float32),
            jax.ShapeDtypeStruct((n_tok, 1), jnp.float32),
        ],
    )(x, Wg)

    eflat = jnp.concatenate([i1, i2], axis=0).reshape(-1)
    wkflat = jnp.concatenate([w1n, w2n], axis=0).reshape(-1)

    mesh = plsc.VectorSubcoreMesh(core_axis_name="c", subcore_axis_name="s")
    route = pl.kernel(
        _route_body,
        compiler_params=pltpu.CompilerParams(needs_layout_passes=False),
        out_type=[
            jax.ShapeDtypeStruct((n_slot, d_model), jnp.float32),
            jax.ShapeDtypeStruct((n_slot,), jnp.float32),
            jax.ShapeDtypeStruct((32,), jnp.int32),
            jax.ShapeDtypeStruct((n_tok,), jnp.int32),
            jax.ShapeDtypeStruct((n_tok,), jnp.int32),
        ],
        mesh=mesh,
        scratch_types=[
            pltpu.VMEM((256,), jnp.int32),
            pltpu.VMEM((512,), jnp.int32),
            pltpu.VMEM_SHARED((512,), jnp.int32),
            pltpu.VMEM((16,), jnp.int32),
            pltpu.VMEM((16,), jnp.int32),
            pltpu.VMEM((128, d_model), jnp.float32),
            pltpu.VMEM((128,), jnp.int32),
            pltpu.VMEM((128,), jnp.float32),
            pltpu.VMEM((32,), jnp.int32),
            pltpu.SemaphoreType.DMA,
            pltpu.SemaphoreType.DMA,
            pltpu.SemaphoreType.DMA,
        ],
    )
    xs, wslot, meta, slot_e, slot_o = route(eflat, wkflat, x)

    ys = pl.pallas_call(
        _ffn_kernel,
        grid_spec=pltpu.PrefetchScalarGridSpec(
            num_scalar_prefetch=2,
            grid=(_MAXB, nhb),
            in_specs=[
                pl.BlockSpec((_BLK, d_model), lambda b, hb, be, nb: (b, 0)),
                pl.BlockSpec((_BLK, 1), lambda b, hb, be, nb: (b, 0)),
                pl.BlockSpec((1, _HB, d_model),
                             lambda b, hb, be, nb: (be[b], hb, 0)),
                pl.BlockSpec((1, _HB, d_model),
                             lambda b, hb, be, nb: (be[b], hb, 0)),
                pl.BlockSpec((1, d_model, _HB),
                             lambda b, hb, be, nb: (be[b], 0, hb)),
            ],
            out_specs=pl.BlockSpec((_BLK, d_model),
                                   lambda b, hb, be, nb: (b, 0)),
        ),
        out_shape=jax.ShapeDtypeStruct((n_slot, d_model), jnp.float32),
    )(meta[:_MAXB], meta[_MAXB:_MAXB + 1], xs, wslot.reshape(n_slot, 1),
      W1, W2, W3)

    combine = pl.kernel(
        _combine_body,
        out_type=jax.ShapeDtypeStruct((n_tok, d_model), jnp.float32),
        mesh=plsc.VectorSubcoreMesh(core_axis_name="c",
                                    subcore_axis_name="s"),
        scratch_types=[
            pltpu.VMEM((64,), jnp.int32),
            pltpu.VMEM((64,), jnp.int32),
            pltpu.VMEM((64, d_model), jnp.float32),
            pltpu.VMEM((64, d_model), jnp.float32),
            pltpu.SemaphoreType.DMA,
            pltpu.SemaphoreType.DMA,
        ],
    )
    out = combine(ys, slot_e, slot_o)
    return out


# speedup vs baseline: 1.3834x; 1.1473x over previous
import jax
import jax.numpy as jnp
from jax import lax
from jax.experimental import pallas as pl
from jax.experimental.pallas import tpu as pltpu
from jax.experimental.pallas import tpu_sc as plsc

_NC = 2
_NS = 16
_BLK = 512
_MAXB = 16
_HB = 512


def _gate_kernel(x_ref, wg_ref, e_ref, w_ref):
    x = x_ref[...]
    wg = wg_ref[...]
    logits = lax.dot_general(x, wg, (((1,), (1,)), ((), ())),
                             preferred_element_type=jnp.float32)
    m = jnp.max(logits, axis=1, keepdims=True)
    p = jnp.exp(logits - m)
    g = p / jnp.sum(p, axis=1, keepdims=True)
    num_e = g.shape[1]
    iota = lax.broadcasted_iota(jnp.int32, g.shape, 1)
    v1 = jnp.max(g, axis=1, keepdims=True)
    i1 = jnp.min(jnp.where(g >= v1, iota, num_e), axis=1, keepdims=True)
    g2 = jnp.where(iota == i1, -1.0, g)
    v2 = jnp.max(g2, axis=1, keepdims=True)
    i2 = jnp.min(jnp.where(g2 >= v2, iota, num_e), axis=1, keepdims=True)
    denom = v1 + v2 + 1e-9
    n = x.shape[0]
    e_ref[pl.ds(0, n), :] = i1
    e_ref[pl.ds(n, n), :] = i2
    w_ref[pl.ds(0, n), :] = v1 / denom
    w_ref[pl.ds(n, n), :] = v2 / denom


def _vgather(src, idx):
    return src.at[idx].get(mode="promise_in_bounds")


_L15 = None


def _splat_last(cs):
    return _vgather(cs, jnp.full((16,), 15, jnp.int32))


def _lane_hist(tv, lanes):
    cnt = jnp.zeros((16,), jnp.int32)
    for e in range(8):
        cs = plsc.cumsum(jnp.where(tv == e, 1, 0))
        cnt = cnt + jnp.where(lanes == e, _splat_last(cs), 0)
    return cnt


def _route_body(eflat, wkflat, x, xs, wslot, meta, slot_e, slot_o,
                tidx2_v, countall_v, shared_counts, cnta_v, cntb_v,
                xv, slots_v, wkv, bev, semx, sem1, sem2):
    c = lax.axis_index("c")
    s = lax.axis_index("s")
    w_rank = 2 * s + c
    ca = 2 * s
    lanes = lax.broadcasted_iota(jnp.int32, (16,), 0)

    tok_base = (w_rank % 16) * 128
    dx = pltpu.async_copy(x.at[pl.ds(tok_base, 128)], xv, semx)

    pltpu.sync_copy(eflat.at[pl.ds(ca * 128, 256)], tidx2_v)

    cnt_a = jnp.zeros((16,), jnp.int32)
    cnt_b = jnp.zeros((16,), jnp.int32)
    for v in range(8):
        cnt_a = cnt_a + _lane_hist(tidx2_v[pl.ds(v * 16, 16)], lanes)
        cnt_b = cnt_b + _lane_hist(tidx2_v[pl.ds(128 + v * 16, 16)], lanes)
    cnta_v[...] = cnt_a
    cntb_v[...] = cnt_b

    pltpu.sync_copy(cnta_v, shared_counts.at[pl.ds(ca * 16, 16)])
    pltpu.sync_copy(cntb_v, shared_counts.at[pl.ds(ca * 16 + 16, 16)])
    plsc.subcore_barrier()
    pltpu.sync_copy(shared_counts, countall_v)

    def accrow(w, carry):
        tot, pre = carry
        row = countall_v[pl.ds(w * 16, 16)]
        return tot + row, pre + jnp.where(w < w_rank, row, 0)

    zero = jnp.zeros((16,), jnp.int32)
    tot, pre = lax.fori_loop(0, 32, accrow, (zero, zero))

    nb_vec = (tot + (_BLK - 1)) // _BLK
    inc = plsc.cumsum(nb_vec)
    excl = inc - nb_vec
    base_vec = excl * _BLK + pre

    bev_vec = jnp.zeros((16,), jnp.int32)
    for e in range(8):
        ince = _vgather(inc, jnp.full((16,), e, jnp.int32))
        bev_vec = bev_vec + jnp.where(ince <= lanes, 1, 0)
    nblocks = _vgather(inc, jnp.full((16,), 7, jnp.int32))
    tail_e = _vgather(bev_vec, jnp.maximum(nblocks - 1, 0))
    bev[pl.ds(0, 16)] = jnp.minimum(bev_vec, tail_e)
    bev[pl.ds(16, 16)] = nblocks

    off = c * 128
    for v in range(8):
        tv = tidx2_v[pl.ds(off + v * 16, 16)]
        rank = jnp.zeros((16,), jnp.int32)
        cnt = jnp.zeros((16,), jnp.int32)
        for e in range(8):
            m = tv == e
            cs = plsc.cumsum(jnp.where(m, 1, 0))
            rank = rank + jnp.where(m, cs - 1, 0)
            cnt = cnt + jnp.where(lanes == e, _splat_last(cs), 0)
        sel = _vgather(base_vec, tv)
        slots_v[pl.ds(v * 16, 16)] = sel + rank
        base_vec = base_vec + cnt

    pltpu.sync_copy(wkflat.at[pl.ds(w_rank * 128, 128)], wkv)
    dx.wait()
    d1 = pltpu.async_copy(xv, xs.at[slots_v], sem1)
    d2 = pltpu.async_copy(wkv, wslot.at[slots_v], sem2)

    @pl.when(w_rank < 16)
    def _():
        pltpu.sync_copy(slots_v, slot_e.at[pl.ds(tok_base, 128)])

    @pl.when(w_rank >= 16)
    def _():
        pltpu.sync_copy(slots_v, slot_o.at[pl.ds(tok_base, 128)])

    @pl.when((c == 0) & (s == 0))
    def _():
        pltpu.sync_copy(bev, meta)

    d1.wait()
    d2.wait()


def _ffn_kernel(be_ref, nb_ref, xs_ref, ws_ref, w1_ref, w2_ref, w3_ref,
                ys_ref):
    b = pl.program_id(0)
    hb = pl.program_id(1)
    nhb = pl.num_programs(1)

    @pl.when(b < nb_ref[0])
    def _():
        xs = xs_ref[...]
        w1 = w1_ref[0]
        w2 = w2_ref[0]
        w3 = w3_ref[0]
        h1 = lax.dot_general(xs, w1, (((1,), (1,)), ((), ())),
                             preferred_element_type=jnp.float32)
        h2 = lax.dot_general(xs, w2, (((1,), (1,)), ((), ())),
                             preferred_element_type=jnp.float32)
        sg = 1.0 / (1.0 + jnp.exp(-h1))
        gmid = h1 * sg * h2
        part = lax.dot_general(gmid, w3, (((1,), (1,)), ((), ())),
                               preferred_element_type=jnp.float32)

        @pl.when(hb == 0)
        def _():
            ys_ref[...] = part

        @pl.when(hb != 0)
        def _():
            ys_ref[...] += part

        @pl.when(hb == nhb - 1)
        def _():
            ys_ref[...] *= ws_ref[...]


def _combine_body(ys, slot_e, slot_o, out, se_v, so_v, ya, yb, sem1, sem2):
    c = lax.axis_index("c")
    s = lax.axis_index("s")
    w = 2 * s + c
    base = w * 64
    pltpu.sync_copy(slot_e.at[pl.ds(base, 64)], se_v)
    pltpu.sync_copy(slot_o.at[pl.ds(base, 64)], so_v)
    g1 = pltpu.async_copy(ys.at[se_v], ya, sem1)
    g2 = pltpu.async_copy(ys.at[so_v], yb, sem2)
    g1.wait()
    g2.wait()

    nvec = ya.shape[1] // 16

    def add_row(j, _):
        for v in range(nvec):
            sl = pl.ds(v * 16, 16)
            ya[j, sl] = ya[j, sl] + yb[j, sl]
        return 0
    lax.fori_loop(0, 64, add_row, 0)
    pltpu.sync_copy(ya, out.at[pl.ds(base, 64)])


def kernel(x, Wg, W1, W2, W3):
    n_tok, d_model = x.shape
    num_e, hidden, _ = W1.shape
    nhb = hidden // _HB
    n_slot = _MAXB * _BLK

    eflat2, wkflat2 = pl.pallas_call(
        _gate_kernel,
        out_shape=[
            jax.ShapeDtypeStruct((2 * n_tok, 1), jnp.int32),
            jax.ShapeDtypeStruct((2 * n_tok, 1), jnp.float32),
        ],
    )(x, Wg)
    eflat = eflat2.reshape(-1)
    wkflat = wkflat2.reshape(-1)

    mesh = plsc.VectorSubcoreMesh(core_axis_name="c", subcore_axis_name="s")
    route = pl.kernel(
        _route_body,
        compiler_params=pltpu.CompilerParams(needs_layout_passes=False),
        out_type=[
            jax.ShapeDtypeStruct((n_slot, d_model), jnp.float32),
            jax.ShapeDtypeStruct((n_slot,), jnp.float32),
            jax.ShapeDtypeStruct((32,), jnp.int32),
            jax.ShapeDtypeStruct((n_tok,), jnp.int32),
            jax.ShapeDtypeStruct((n_tok,), jnp.int32),
        ],
        mesh=mesh,
        scratch_types=[
            pltpu.VMEM((256,), jnp.int32),
            pltpu.VMEM((512,), jnp.int32),
            pltpu.VMEM_SHARED((512,), jnp.int32),
            pltpu.VMEM((16,), jnp.int32),
            pltpu.VMEM((16,), jnp.int32),
            pltpu.VMEM((128, d_model), jnp.float32),
            pltpu.VMEM((128,), jnp.int32),
            pltpu.VMEM((128,), jnp.float32),
            pltpu.VMEM((32,), jnp.int32),
            pltpu.SemaphoreType.DMA,
            pltpu.SemaphoreType.DMA,
            pltpu.SemaphoreType.DMA,
        ],
    )
    xs, wslot, meta, slot_e, slot_o = route(eflat, wkflat, x)

    def _row_idx(b, hb, be, nb):
        return (jnp.minimum(b, nb[0] - 1), 0)

    def _w12_idx(b, hb, be, nb):
        return (be[b], jnp.where(b < nb[0], hb, nhb - 1), 0)

    def _w3_idx(b, hb, be, nb):
        return (be[b], 0, jnp.where(b < nb[0], hb, nhb - 1))

    ys = pl.pallas_call(
        _ffn_kernel,
        grid_spec=pltpu.PrefetchScalarGridSpec(
            num_scalar_prefetch=2,
            grid=(_MAXB, nhb),
            in_specs=[
                pl.BlockSpec((_BLK, d_model), _row_idx),
                pl.BlockSpec((_BLK, 1), _row_idx),
                pl.BlockSpec((1, _HB, d_model), _w12_idx),
                pl.BlockSpec((1, _HB, d_model), _w12_idx),
                pl.BlockSpec((1, d_model, _HB), _w3_idx),
            ],
            out_specs=pl.BlockSpec((_BLK, d_model), _row_idx),
        ),
        out_shape=jax.ShapeDtypeStruct((n_slot, d_model), jnp.float32),
    )(meta[:_MAXB], meta[_MAXB:_MAXB + 1], xs, wslot.reshape(n_slot, 1),
      W1, W2, W3)

    combine = pl.kernel(
        _combine_body,
        out_type=jax.ShapeDtypeStruct((n_tok, d_model), jnp.float32),
        mesh=plsc.VectorSubcoreMesh(core_axis_name="c",
                                    subcore_axis_name="s"),
        scratch_types=[
            pltpu.VMEM((64,), jnp.int32),
            pltpu.VMEM((64,), jnp.int32),
            pltpu.VMEM((64, d_model), jnp.float32),
            pltpu.VMEM((64, d_model), jnp.float32),
            pltpu.SemaphoreType.DMA,
            pltpu.SemaphoreType.DMA,
        ],
    )
    out = combine(ys, slot_e, slot_o)
    return out


# speedup vs baseline: 1.4024x; 1.0138x over previous
import jax
import jax.numpy as jnp
from jax import lax
from jax.experimental import pallas as pl
from jax.experimental.pallas import tpu as pltpu
from jax.experimental.pallas import tpu_sc as plsc

_NC = 2
_NS = 16
_BLK = 512
_MAXB = 16
_HB = 512


def _gate_kernel(x_ref, wg_ref, e_ref, w_ref):
    x = x_ref[...]
    wg = wg_ref[...]
    logits = lax.dot_general(x, wg, (((1,), (1,)), ((), ())),
                             preferred_element_type=jnp.float32)
    m = jnp.max(logits, axis=1, keepdims=True)
    p = jnp.exp(logits - m)
    g = p / jnp.sum(p, axis=1, keepdims=True)
    num_e = g.shape[1]
    iota = lax.broadcasted_iota(jnp.int32, g.shape, 1)
    v1 = jnp.max(g, axis=1, keepdims=True)
    i1 = jnp.min(jnp.where(g >= v1, iota, num_e), axis=1, keepdims=True)
    g2 = jnp.where(iota == i1, -1.0, g)
    v2 = jnp.max(g2, axis=1, keepdims=True)
    i2 = jnp.min(jnp.where(g2 >= v2, iota, num_e), axis=1, keepdims=True)
    denom = v1 + v2 + 1e-9
    n = x.shape[0]
    e_ref[pl.ds(0, n), :] = i1
    e_ref[pl.ds(n, n), :] = i2
    w_ref[pl.ds(0, n), :] = v1 / denom
    w_ref[pl.ds(n, n), :] = v2 / denom


def _vgather(src, idx):
    return src.at[idx].get(mode="promise_in_bounds")


_L15 = None


def _splat_last(cs):
    return _vgather(cs, jnp.full((16,), 15, jnp.int32))


def _lane_hist(tv, lanes):
    cnt = jnp.zeros((16,), jnp.int32)
    for e in range(8):
        cs = plsc.cumsum(jnp.where(tv == e, 1, 0))
        cnt = cnt + jnp.where(lanes == e, _splat_last(cs), 0)
    return cnt


def _route_body(eflat, wkflat, x, xs, wslot, meta, slot_e, slot_o,
                tidx2_v, countall_v, shared_counts, cnta_v, cntb_v,
                xv, slots_v, wkv, bev, semx, sem1, sem2):
    c = lax.axis_index("c")
    s = lax.axis_index("s")
    w_rank = 2 * s + c
    ca = 2 * s
    lanes = lax.broadcasted_iota(jnp.int32, (16,), 0)

    tok_base = (w_rank % 16) * 128
    dx = pltpu.async_copy(x.at[pl.ds(tok_base, 128)], xv, semx)

    pltpu.sync_copy(eflat.at[pl.ds(ca * 128, 256)], tidx2_v)

    cnt_a = jnp.zeros((16,), jnp.int32)
    cnt_b = jnp.zeros((16,), jnp.int32)
    for v in range(8):
        cnt_a = cnt_a + _lane_hist(tidx2_v[pl.ds(v * 16, 16)], lanes)
        cnt_b = cnt_b + _lane_hist(tidx2_v[pl.ds(128 + v * 16, 16)], lanes)
    cnta_v[...] = cnt_a
    cntb_v[...] = cnt_b

    pltpu.sync_copy(cnta_v, shared_counts.at[pl.ds(ca * 16, 16)])
    pltpu.sync_copy(cntb_v, shared_counts.at[pl.ds(ca * 16 + 16, 16)])
    plsc.subcore_barrier()
    pltpu.sync_copy(shared_counts, countall_v)

    def accrow(w, carry):
        tot, pre = carry
        row = countall_v[pl.ds(w * 16, 16)]
        return tot + row, pre + jnp.where(w < w_rank, row, 0)

    zero = jnp.zeros((16,), jnp.int32)
    tot, pre = lax.fori_loop(0, 32, accrow, (zero, zero))

    nb_vec = (tot + (_BLK - 1)) // _BLK
    inc = plsc.cumsum(nb_vec)
    excl = inc - nb_vec
    base_vec = excl * _BLK + pre

    bev_vec = jnp.zeros((16,), jnp.int32)
    for e in range(8):
        ince = _vgather(inc, jnp.full((16,), e, jnp.int32))
        bev_vec = bev_vec + jnp.where(ince <= lanes, 1, 0)
    nblocks = _vgather(inc, jnp.full((16,), 7, jnp.int32))
    tail_e = _vgather(bev_vec, jnp.maximum(nblocks - 1, 0))
    bev[pl.ds(0, 16)] = jnp.minimum(bev_vec, tail_e)
    bev[pl.ds(16, 16)] = nblocks

    off = c * 128
    for v in range(8):
        tv = tidx2_v[pl.ds(off + v * 16, 16)]
        rank = jnp.zeros((16,), jnp.int32)
        cnt = jnp.zeros((16,), jnp.int32)
        for e in range(8):
            m = tv == e
            cs = plsc.cumsum(jnp.where(m, 1, 0))
            rank = rank + jnp.where(m, cs - 1, 0)
            cnt = cnt + jnp.where(lanes == e, _splat_last(cs), 0)
        sel = _vgather(base_vec, tv)
        slots_v[pl.ds(v * 16, 16)] = sel + rank
        base_vec = base_vec + cnt

    pltpu.sync_copy(wkflat.at[pl.ds(w_rank * 128, 128)], wkv)
    dx.wait()
    d1 = pltpu.async_copy(xv, xs.at[slots_v], sem1)
    d2 = pltpu.async_copy(wkv, wslot.at[slots_v], sem2)

    @pl.when(w_rank < 16)
    def _():
        pltpu.sync_copy(slots_v, slot_e.at[pl.ds(tok_base, 128)])

    @pl.when(w_rank >= 16)
    def _():
        pltpu.sync_copy(slots_v, slot_o.at[pl.ds(tok_base, 128)])

    @pl.when((c == 0) & (s == 0))
    def _():
        pltpu.sync_copy(bev, meta)

    d1.wait()
    d2.wait()


def _ffn_kernel(m_ref, xs_ref, ws_ref, w1_ref, w2_ref, w3_ref, ys_ref):
    b = pl.program_id(0)
    hb = pl.program_id(1)
    nhb = pl.num_programs(1)

    @pl.when(b < m_ref[_MAXB])
    def _():
        xs = xs_ref[...]
        w1 = w1_ref[0]
        w2 = w2_ref[0]
        w3 = w3_ref[0]
        h1 = lax.dot_general(xs, w1, (((1,), (1,)), ((), ())),
                             preferred_element_type=jnp.float32)
        h2 = lax.dot_general(xs, w2, (((1,), (1,)), ((), ())),
                             preferred_element_type=jnp.float32)
        sg = 1.0 / (1.0 + jnp.exp(-h1))
        gmid = h1 * sg * h2
        part = lax.dot_general(gmid, w3, (((1,), (1,)), ((), ())),
                               preferred_element_type=jnp.float32)

        @pl.when(hb == 0)
        def _():
            ys_ref[...] = part

        @pl.when(hb != 0)
        def _():
            ys_ref[...] += part

        @pl.when(hb == nhb - 1)
        def _():
            ys_ref[...] *= ws_ref[...]


def _combine_body(ys, slot_e, slot_o, out, se_v, so_v, ya, yb, sem1, sem2):
    c = lax.axis_index("c")
    s = lax.axis_index("s")
    w = 2 * s + c
    base = w * 64
    pltpu.sync_copy(slot_e.at[pl.ds(base, 64)], se_v)
    pltpu.sync_copy(slot_o.at[pl.ds(base, 64)], so_v)
    g1 = pltpu.async_copy(ys.at[se_v], ya, sem1)
    g2 = pltpu.async_copy(ys.at[so_v], yb, sem2)
    g1.wait()
    g2.wait()

    nvec = ya.shape[1] // 16

    def add_row(j, _):
        for v in range(nvec):
            sl = pl.ds(v * 16, 16)
            ya[j, sl] = ya[j, sl] + yb[j, sl]
        return 0
    lax.fori_loop(0, 64, add_row, 0)
    pltpu.sync_copy(ya, out.at[pl.ds(base, 64)])


def kernel(x, Wg, W1, W2, W3):
    n_tok, d_model = x.shape
    num_e, hidden, _ = W1.shape
    nhb = hidden // _HB
    n_slot = _MAXB * _BLK

    eflat2, wkflat2 = pl.pallas_call(
        _gate_kernel,
        out_shape=[
            jax.ShapeDtypeStruct((2 * n_tok, 1), jnp.int32),
            jax.ShapeDtypeStruct((2 * n_tok, 1), jnp.float32),
        ],
    )(x, Wg)
    eflat = eflat2.reshape(-1)
    wkflat = wkflat2.reshape(-1)

    mesh = plsc.VectorSubcoreMesh(core_axis_name="c", subcore_axis_name="s")
    route = pl.kernel(
        _route_body,
        compiler_params=pltpu.CompilerParams(needs_layout_passes=False),
        out_type=[
            jax.ShapeDtypeStruct((n_slot, d_model), jnp.float32),
            jax.ShapeDtypeStruct((n_slot,), jnp.float32),
            jax.ShapeDtypeStruct((32,), jnp.int32),
            jax.ShapeDtypeStruct((n_tok,), jnp.int32),
            jax.ShapeDtypeStruct((n_tok,), jnp.int32),
        ],
        mesh=mesh,
        scratch_types=[
            pltpu.VMEM((256,), jnp.int32),
            pltpu.VMEM((512,), jnp.int32),
            pltpu.VMEM_SHARED((512,), jnp.int32),
            pltpu.VMEM((16,), jnp.int32),
            pltpu.VMEM((16,), jnp.int32),
            pltpu.VMEM((128, d_model), jnp.float32),
            pltpu.VMEM((128,), jnp.int32),
            pltpu.VMEM((128,), jnp.float32),
            pltpu.VMEM((32,), jnp.int32),
            pltpu.SemaphoreType.DMA,
            pltpu.SemaphoreType.DMA,
            pltpu.SemaphoreType.DMA,
        ],
    )
    xs, wslot, meta, slot_e, slot_o = route(eflat, wkflat, x)

    def _row_idx(b, hb, m):
        return (jnp.minimum(b, m[_MAXB] - 1), 0)

    def _w12_idx(b, hb, m):
        return (m[b], jnp.where(b < m[_MAXB], hb, nhb - 1), 0)

    def _w3_idx(b, hb, m):
        return (m[b], 0, jnp.where(b < m[_MAXB], hb, nhb - 1))

    ys = pl.pallas_call(
        _ffn_kernel,
        grid_spec=pltpu.PrefetchScalarGridSpec(
            num_scalar_prefetch=1,
            grid=(_MAXB, nhb),
            in_specs=[
                pl.BlockSpec((_BLK, d_model), _row_idx),
                pl.BlockSpec((_BLK, 1), _row_idx),
                pl.BlockSpec((1, _HB, d_model), _w12_idx),
                pl.BlockSpec((1, _HB, d_model), _w12_idx),
                pl.BlockSpec((1, d_model, _HB), _w3_idx),
            ],
            out_specs=pl.BlockSpec((_BLK, d_model), _row_idx),
        ),
        out_shape=jax.ShapeDtypeStruct((n_slot, d_model), jnp.float32),
    )(meta, xs, wslot.reshape(n_slot, 1), W1, W2, W3)

    combine = pl.kernel(
        _combine_body,
        out_type=jax.ShapeDtypeStruct((n_tok, d_model), jnp.float32),
        mesh=plsc.VectorSubcoreMesh(core_axis_name="c",
                                    subcore_axis_name="s"),
        scratch_types=[
            pltpu.VMEM((64,), jnp.int32),
            pltpu.VMEM((64,), jnp.int32),
            pltpu.VMEM((64, d_model), jnp.float32),
            pltpu.VMEM((64, d_model), jnp.float32),
            pltpu.SemaphoreType.DMA,
            pltpu.SemaphoreType.DMA,
        ],
    )
    out = combine(ys, slot_e, slot_o)
    return out


# speedup vs baseline: 1.8282x; 1.3036x over previous
import jax
import jax.numpy as jnp
from jax import lax
from jax.experimental import pallas as pl
from jax.experimental.pallas import tpu as pltpu
from jax.experimental.pallas import tpu_sc as plsc

_NC = 2
_NS = 16
_BLK = 576
_MAXB = 14
_HB = 512


def _gate_kernel(x_ref, wg_ref, e_ref, w_ref):
    x = x_ref[...]
    wg = wg_ref[...]
    logits = lax.dot_general(x, wg, (((1,), (1,)), ((), ())),
                             preferred_element_type=jnp.float32)
    m = jnp.max(logits, axis=1, keepdims=True)
    p = jnp.exp(logits - m)
    g = p / jnp.sum(p, axis=1, keepdims=True)
    num_e = g.shape[1]
    iota = lax.broadcasted_iota(jnp.int32, g.shape, 1)
    v1 = jnp.max(g, axis=1, keepdims=True)
    i1 = jnp.min(jnp.where(g >= v1, iota, num_e), axis=1, keepdims=True)
    g2 = jnp.where(iota == i1, -1.0, g)
    v2 = jnp.max(g2, axis=1, keepdims=True)
    i2 = jnp.min(jnp.where(g2 >= v2, iota, num_e), axis=1, keepdims=True)
    denom = v1 + v2 + 1e-9
    n = x.shape[0]
    e_ref[pl.ds(0, n), :] = i1
    e_ref[pl.ds(n, n), :] = i2
    w_ref[pl.ds(0, n), :] = v1 / denom
    w_ref[pl.ds(n, n), :] = v2 / denom


def _vgather(src, idx):
    return src.at[idx].get(mode="promise_in_bounds")


_L15 = None


def _splat_last(cs):
    return _vgather(cs, jnp.full((16,), 15, jnp.int32))


def _lane_hist(tv, lanes):
    cnt = jnp.zeros((16,), jnp.int32)
    for e in range(8):
        cs = plsc.cumsum(jnp.where(tv == e, 1, 0))
        cnt = cnt + jnp.where(lanes == e, _splat_last(cs), 0)
    return cnt


def _route_body(eflat, wkflat, x, xs, wslot, meta, slot_e, slot_o,
                tidx2_v, countall_v, shared_counts, cnta_v, cntb_v,
                xv, slots_v, wkv, bev, semx, sem1, sem2):
    c = lax.axis_index("c")
    s = lax.axis_index("s")
    w_rank = 2 * s + c
    ca = 2 * s
    lanes = lax.broadcasted_iota(jnp.int32, (16,), 0)

    tok_base = (w_rank % 16) * 128
    dx = pltpu.async_copy(x.at[pl.ds(tok_base, 128)], xv, semx)

    pltpu.sync_copy(eflat.at[pl.ds(ca * 128, 256)], tidx2_v)

    cnt_a = jnp.zeros((16,), jnp.int32)
    cnt_b = jnp.zeros((16,), jnp.int32)
    for v in range(8):
        cnt_a = cnt_a + _lane_hist(tidx2_v[pl.ds(v * 16, 16)], lanes)
        cnt_b = cnt_b + _lane_hist(tidx2_v[pl.ds(128 + v * 16, 16)], lanes)
    cnta_v[...] = cnt_a
    cntb_v[...] = cnt_b

    pltpu.sync_copy(cnta_v, shared_counts.at[pl.ds(ca * 16, 16)])
    pltpu.sync_copy(cntb_v, shared_counts.at[pl.ds(ca * 16 + 16, 16)])
    plsc.subcore_barrier()
    pltpu.sync_copy(shared_counts, countall_v)

    def accrow(w, carry):
        tot, pre = carry
        row = countall_v[pl.ds(w * 16, 16)]
        return tot + row, pre + jnp.where(w < w_rank, row, 0)

    zero = jnp.zeros((16,), jnp.int32)
    tot, pre = lax.fori_loop(0, 32, accrow, (zero, zero))

    nb_vec = (tot + (_BLK - 1)) // _BLK
    inc = plsc.cumsum(nb_vec)
    excl = inc - nb_vec
    base_vec = excl * _BLK + pre

    bev_vec = jnp.zeros((16,), jnp.int32)
    for e in range(8):
        ince = _vgather(inc, jnp.full((16,), e, jnp.int32))
        bev_vec = bev_vec + jnp.where(ince <= lanes, 1, 0)
    nblocks = _vgather(inc, jnp.full((16,), 7, jnp.int32))
    tail_e = _vgather(bev_vec, jnp.maximum(nblocks - 1, 0))
    bev[pl.ds(0, 16)] = jnp.minimum(bev_vec, tail_e)
    bev[pl.ds(16, 16)] = nblocks

    off = c * 128
    for v in range(8):
        tv = tidx2_v[pl.ds(off + v * 16, 16)]
        rank = jnp.zeros((16,), jnp.int32)
        cnt = jnp.zeros((16,), jnp.int32)
        for e in range(8):
            m = tv == e
            cs = plsc.cumsum(jnp.where(m, 1, 0))
            rank = rank + jnp.where(m, cs - 1, 0)
            cnt = cnt + jnp.where(lanes == e, _splat_last(cs), 0)
        sel = _vgather(base_vec, tv)
        slots_v[pl.ds(v * 16, 16)] = sel + rank
        base_vec = base_vec + cnt

    pltpu.sync_copy(wkflat.at[pl.ds(w_rank * 128, 128)], wkv)
    dx.wait()
    d1 = pltpu.async_copy(xv, xs.at[slots_v], sem1)
    d2 = pltpu.async_copy(wkv, wslot.at[slots_v], sem2)

    @pl.when(w_rank < 16)
    def _():
        pltpu.sync_copy(slots_v, slot_e.at[pl.ds(tok_base, 128)])

    @pl.when(w_rank >= 16)
    def _():
        pltpu.sync_copy(slots_v, slot_o.at[pl.ds(tok_base, 128)])

    @pl.when((c == 0) & (s == 0))
    def _():
        pltpu.sync_copy(bev, meta)

    d1.wait()
    d2.wait()


def _ffn_kernel(m_ref, xs_ref, ws_ref, w1_ref, w2_ref, w3_ref, ys_ref):
    b = pl.program_id(0)
    hb = pl.program_id(1)
    nhb = pl.num_programs(1)

    @pl.when(b < m_ref[_MAXB])
    def _():
        xs = xs_ref[...]
        w1 = w1_ref[0]
        w2 = w2_ref[0]
        w3 = w3_ref[0]
        h1 = lax.dot_general(xs, w1, (((1,), (1,)), ((), ())),
                             preferred_element_type=jnp.float32)
        h2 = lax.dot_general(xs, w2, (((1,), (1,)), ((), ())),
                             preferred_element_type=jnp.float32)
        sg = 1.0 / (1.0 + jnp.exp(-h1))
        gmid = h1 * sg * h2
        part = lax.dot_general(gmid, w3, (((1,), (1,)), ((), ())),
                               preferred_element_type=jnp.float32)

        @pl.when(hb == 0)
        def _():
            ys_ref[...] = part

        @pl.when(hb != 0)
        def _():
            ys_ref[...] += part

        @pl.when(hb == nhb - 1)
        def _():
            ys_ref[...] *= ws_ref[...]


def _combine_body(ys, slot_e, slot_o, out, se_v, so_v, ya, yb, sem1, sem2):
    c = lax.axis_index("c")
    s = lax.axis_index("s")
    w = 2 * s + c
    base = w * 64
    pltpu.sync_copy(slot_e.at[pl.ds(base, 64)], se_v)
    pltpu.sync_copy(slot_o.at[pl.ds(base, 64)], so_v)
    g1 = pltpu.async_copy(ys.at[se_v], ya, sem1)
    g2 = pltpu.async_copy(ys.at[so_v], yb, sem2)
    g1.wait()
    g2.wait()

    nvec = ya.shape[1] // 16

    def add_row(j, _):
        for v in range(nvec):
            sl = pl.ds(v * 16, 16)
            ya[j, sl] = ya[j, sl] + yb[j, sl]
        return 0
    lax.fori_loop(0, 64, add_row, 0)
    pltpu.sync_copy(ya, out.at[pl.ds(base, 64)])


def kernel(x, Wg, W1, W2, W3):
    n_tok, d_model = x.shape
    num_e, hidden, _ = W1.shape
    nhb = hidden // _HB
    n_slot = _MAXB * _BLK

    eflat2, wkflat2 = pl.pallas_call(
        _gate_kernel,
        out_shape=[
            jax.ShapeDtypeStruct((2 * n_tok, 1), jnp.int32),
            jax.ShapeDtypeStruct((2 * n_tok, 1), jnp.float32),
        ],
    )(x, Wg)
    eflat = eflat2.reshape(-1)
    wkflat = wkflat2.reshape(-1)

    mesh = plsc.VectorSubcoreMesh(core_axis_name="c", subcore_axis_name="s")
    route = pl.kernel(
        _route_body,
        compiler_params=pltpu.CompilerParams(needs_layout_passes=False),
        out_type=[
            jax.ShapeDtypeStruct((n_slot, d_model), jnp.float32),
            jax.ShapeDtypeStruct((n_slot,), jnp.float32),
            jax.ShapeDtypeStruct((32,), jnp.int32),
            jax.ShapeDtypeStruct((n_tok,), jnp.int32),
            jax.ShapeDtypeStruct((n_tok,), jnp.int32),
        ],
        mesh=mesh,
        scratch_types=[
            pltpu.VMEM((256,), jnp.int32),
            pltpu.VMEM((512,), jnp.int32),
            pltpu.VMEM_SHARED((512,), jnp.int32),
            pltpu.VMEM((16,), jnp.int32),
            pltpu.VMEM((16,), jnp.int32),
            pltpu.VMEM((128, d_model), jnp.float32),
            pltpu.VMEM((128,), jnp.int32),
            pltpu.VMEM((128,), jnp.float32),
            pltpu.VMEM((32,), jnp.int32),
            pltpu.SemaphoreType.DMA,
            pltpu.SemaphoreType.DMA,
            pltpu.SemaphoreType.DMA,
        ],
    )
    xs, wslot, meta, slot_e, slot_o = route(eflat, wkflat, x)

    def _row_idx(b, hb, m):
        return (jnp.minimum(b, m[_MAXB] - 1), 0)

    def _w12_idx(b, hb, m):
        return (m[b], jnp.where(b < m[_MAXB], hb, nhb - 1), 0)

    def _w3_idx(b, hb, m):
        return (m[b], 0, jnp.where(b < m[_MAXB], hb, nhb - 1))

    ys = pl.pallas_call(
        _ffn_kernel,
        grid_spec=pltpu.PrefetchScalarGridSpec(
            num_scalar_prefetch=1,
            grid=(_MAXB, nhb),
            in_specs=[
                pl.BlockSpec((_BLK, d_model), _row_idx),
                pl.BlockSpec((_BLK, 1), _row_idx),
                pl.BlockSpec((1, _HB, d_model), _w12_idx),
                pl.BlockSpec((1, _HB, d_model), _w12_idx),
                pl.BlockSpec((1, d_model, _HB), _w3_idx),
            ],
            out_specs=pl.BlockSpec((_BLK, d_model), _row_idx),
        ),
        out_shape=jax.ShapeDtypeStruct((n_slot, d_model), jnp.float32),
    )(meta, xs, wslot.reshape(n_slot, 1), W1, W2, W3)

    combine = pl.kernel(
        _combine_body,
        out_type=jax.ShapeDtypeStruct((n_tok, d_model), jnp.float32),
        mesh=plsc.VectorSubcoreMesh(core_axis_name="c",
                                    subcore_axis_name="s"),
        scratch_types=[
            pltpu.VMEM((64,), jnp.int32),
            pltpu.VMEM((64,), jnp.int32),
            pltpu.VMEM((64, d_model), jnp.float32),
            pltpu.VMEM((64, d_model), jnp.float32),
            pltpu.SemaphoreType.DMA,
            pltpu.SemaphoreType.DMA,
        ],
    )
    out = combine(ys, slot_e, slot_o)
    return out
